# baseline (speedup 1.0000x reference)
import jax
import jax.numpy as jnp
from jax.experimental import pallas as pl
from jax.experimental.pallas import tpu as pltpu

N_BLOCKS = 3
N_ATOM_BASIS = 128
N_FILTERS = 64
N_GAUSS = 25
MAX_Z = 5
CUTOFF = 5.0
N_ATOMS = 128

_DELTA = CUTOFF / (N_GAUSS - 1)
_ALPHA = 0.5 / (_DELTA * _DELTA)

_TI = 32
_NT = N_ATOMS // _TI
_NP = _NT // 2

_F32 = jnp.float32


def _dot(a, b, dn=(((1,), (0,)), ((), ()))):
    return jax.lax.dot_general(a, b, dn, preferred_element_type=_F32)


def _body(pos_ref, an_ref, emb_ref, unf_ref, ct_ref,
          w1a_ref, b1a_ref, w2p_ref, b2d_ref,
          i2fw_ref, i2fb_ref, f2ow_ref, f2ob_ref, dw_ref, db_ref,
          e1w_ref, e1b_ref, e2w_ref, e2b_ref, out_ref, w_scr, f_scr):
    A = N_ATOMS
    F = N_FILTERS
    TP = _TI * A

    pos = pos_ref[0]
    dot = _dot(pos, pos, (((1,), (1,)), ((), ())))
    psq = pos * pos
    n2c = jnp.sum(psq, axis=1, keepdims=True)
    ones13 = jnp.ones((1, 3), _F32)
    n2r = _dot(ones13, psq, (((1,), (1,)), ((), ())))
    r2 = n2c + n2r - 2.0 * dot
    ii = jax.lax.broadcasted_iota(jnp.int32, (A, A), 0)
    jj = jax.lax.broadcasted_iota(jnp.int32, (A, A), 1)
    r2 = jnp.where(ii == jj, 0.0, jnp.maximum(r2, 0.0))
    r = jnp.sqrt(r2)

    rrep = jnp.broadcast_to(r[:, None, :], (A, _TI, A))
    m1 = (jax.lax.broadcasted_iota(jnp.int32, (1, _TI, A), 1) ==
          jax.lax.broadcasted_iota(jnp.int32, (1, _TI, A), 2) % _TI
          ).astype(_F32)
    rm = (rrep * m1).reshape(TP, A)
    runf = _dot(rm, unf_ref[...])
    fq = jnp.exp(-_ALPHA * (runf - ct_ref[...]) ** 2)
    f_scr[...] = fq

    def filt_l1(b):
        return jnp.tanh(_dot(f_scr[...], w1a_ref[b]) + b1a_ref[b])

    def filt_l2(h, b, u):
        wp = _dot(h[:, 128 * u:128 * (u + 1)], w2p_ref[b]) + b2d_ref[b]
        w_scr[b, u] = wp

    h0b = filt_l1(0)
    for u in range(_NP):
        filt_l2(h0b, 0, u)

    f0a = jnp.exp(-_ALPHA * ct_ref[...] ** 2)
    w_self = []
    for b in range(N_BLOCKS):
        h0 = jnp.tanh(_dot(f0a, w1a_ref[b]) + b1a_ref[b])
        u0 = _dot(h0[:, :128], w2p_ref[b]) + b2d_ref[b]
        w_self.append(u0[:, :F])

    an = an_ref[0]
    onehot = (an == jax.lax.broadcasted_iota(jnp.int32, (A, MAX_Z), 1)
              ).astype(_F32)
    x = _dot(onehot, emb_ref[...])

    for b in range(N_BLOCKS):
        y = _dot(x, i2fw_ref[b]) + i2fb_ref[b]
        y2l = jnp.concatenate([y, y], axis=1)[:, None, :]
        hb = filt_l1(b + 1) if b + 1 < N_BLOCKS else None
        accs = [None] * _NT
        for u in range(_NP):
            z = w_scr[b, u].reshape(A, _TI, 2 * F) * y2l
            n = A
            while n > 1:
                n //= 2
                z = z[:n] + z[n:]
            zz = z.reshape(_TI, 2 * F)
            accs[2 * u] = zz[:, :F]
            accs[2 * u + 1] = zz[:, F:]
            if hb is not None:
                filt_l2(hb, b + 1, u)
        agg = jnp.concatenate(accs, axis=0) - w_self[b] * y
        yo = jnp.tanh(_dot(agg, f2ow_ref[b]) + f2ob_ref[b])
        x = x + _dot(yo, dw_ref[b]) + db_ref[b]

    t1 = jnp.tanh(_dot(x, e1w_ref[...]) + e1b_ref[...])
    o = _dot(t1, e2w_ref[...]) + e2b_ref[...]
    out_ref[0] = jnp.sum(o, axis=0, keepdims=True)


@jax.jit
def kernel(atomic_positions, atomic_numbers, emb, fw1_W, fw1_b, fw2_W, fw2_b,
           in2f_W, in2f_b, f2out_W, f2out_b, dense_W, dense_b,
           e1_W, e1_b, e2_W, e2_b):
    C, A = atomic_positions.shape[0], atomic_positions.shape[1]
    F, G = N_FILTERS, N_GAUSS
    an3 = atomic_numbers.astype(jnp.int32).reshape(C, A, 1)

    w1a = jnp.zeros((N_BLOCKS, _NT * G, _NT * F), _F32)
    for t in range(_NT):
        w1a = w1a.at[:, G * t:G * (t + 1), F * t:F * (t + 1)].set(fw1_W)
    b1a = jnp.tile(fw1_b, (1, _NT))
    w2p = jnp.zeros((N_BLOCKS, 2 * F, 2 * F), _F32)
    w2p = w2p.at[:, :F, :F].set(fw2_W).at[:, F:, F:].set(fw2_W)
    b2d = jnp.tile(fw2_b, (1, 2))
    unf = (jnp.arange(A)[:, None] // _TI ==
           jnp.arange(_NT * G)[None, :] // G).astype(_F32)
    centers = jnp.linspace(0.0, CUTOFF, G, dtype=_F32)
    ctile = jnp.tile(centers, _NT)[None, :]

    def rep(shape):
        nd = len(shape)
        return pl.BlockSpec(shape, lambda c, _n=nd: (0,) * _n)

    in_specs = [
        pl.BlockSpec((1, A, 3), lambda c: (c, 0, 0)),
        pl.BlockSpec((1, A, 1), lambda c: (c, 0, 0)),
        rep(emb.shape),
        rep(unf.shape), rep(ctile.shape),
        rep(w1a.shape), rep(b1a.shape),
        rep(w2p.shape), rep(b2d.shape),
        rep(in2f_W.shape), rep(in2f_b.shape),
        rep(f2out_W.shape), rep(f2out_b.shape),
        rep(dense_W.shape), rep(dense_b.shape),
        rep(e1_W.shape), rep(e1_b.shape),
        rep(e2_W.shape), rep(e2_b.shape),
    ]
    out = pl.pallas_call(
        _body,
        grid=(C,),
        in_specs=in_specs,
        out_specs=pl.BlockSpec((1, 1, 1), lambda c: (c, 0, 0)),
        out_shape=jax.ShapeDtypeStruct((C, 1, 1), _F32),
        scratch_shapes=[pltpu.VMEM((N_BLOCKS, _NP, _TI * A, 2 * F), _F32),
                        pltpu.VMEM((_TI * A, _NT * G), _F32)],
        compiler_params=pltpu.CompilerParams(
            dimension_semantics=("arbitrary",)),
    )(atomic_positions, an3, emb, unf, ctile, w1a, b1a, w2p, b2d,
      in2f_W, in2f_b, f2out_W, f2out_b, dense_W, dense_b,
      e1_W, e1_b, e2_W, e2_b)
    return out.reshape(C, 1)


# speedup vs baseline: 1.0067x; 1.0067x over previous
import jax
import jax.numpy as jnp
from jax.experimental import pallas as pl
from jax.experimental.pallas import tpu as pltpu

N_BLOCKS = 3
N_ATOM_BASIS = 128
N_FILTERS = 64
N_GAUSS = 25
MAX_Z = 5
CUTOFF = 5.0
N_ATOMS = 128

_DELTA = CUTOFF / (N_GAUSS - 1)
_ALPHA = 0.5 / (_DELTA * _DELTA)

_TI = 32
_NT = N_ATOMS // _TI
_NP = _NT // 2

_F32 = jnp.float32


def _dot(a, b, dn=(((1,), (0,)), ((), ()))):
    return jax.lax.dot_general(a, b, dn, preferred_element_type=_F32)


def _body(pos_ref, an_ref, emb_ref, unf_ref, ct_ref,
          w1a_ref, b1a_ref, w2p_ref, b2d_ref,
          i2fw_ref, i2fb_ref, f2ow_ref, f2ob_ref, dw_ref, db_ref,
          e1w_ref, e1b_ref, e2w_ref, e2b_ref, out_ref, w_scr, f_scr):
    A = N_ATOMS
    F = N_FILTERS
    TP = _TI * A

    pos = pos_ref[0]
    dot = _dot(pos, pos, (((1,), (1,)), ((), ())))
    psq = pos * pos
    n2c = jnp.sum(psq, axis=1, keepdims=True)
    ones13 = jnp.ones((1, 3), _F32)
    n2r = _dot(ones13, psq, (((1,), (1,)), ((), ())))
    r2 = n2c + n2r - 2.0 * dot
    ii = jax.lax.broadcasted_iota(jnp.int32, (A, A), 0)
    jj = jax.lax.broadcasted_iota(jnp.int32, (A, A), 1)
    r2 = jnp.where(ii == jj, 0.0, jnp.maximum(r2, 0.0))
    r = jnp.sqrt(r2)

    rrep = jnp.broadcast_to(r[:, None, :], (A, _TI, A))
    m1 = (jax.lax.broadcasted_iota(jnp.int32, (1, _TI, A), 1) ==
          jax.lax.broadcasted_iota(jnp.int32, (1, _TI, A), 2) % _TI
          ).astype(_F32)
    rm = (rrep * m1).reshape(TP, A)
    runf = _dot(rm, unf_ref[...])
    fq = jnp.exp(-_ALPHA * (runf - ct_ref[...]) ** 2)
    f_scr[...] = fq.astype(jnp.bfloat16)

    def filt_l1(b):
        return jnp.tanh(_dot(f_scr[...], w1a_ref[b]) + b1a_ref[b])

    def filt_l2(h, b, u):
        wp = _dot(h[:, 128 * u:128 * (u + 1)], w2p_ref[b]) + b2d_ref[b]
        w_scr[b, u] = wp

    h0b = filt_l1(0)
    for u in range(_NP):
        filt_l2(h0b, 0, u)

    f0a = jnp.exp(-_ALPHA * ct_ref[...] ** 2)
    w_self = []
    for b in range(N_BLOCKS):
        h0 = jnp.tanh(_dot(f0a, w1a_ref[b]) + b1a_ref[b])
        u0 = _dot(h0[:, :128], w2p_ref[b]) + b2d_ref[b]
        w_self.append(u0[:, :F])

    an = an_ref[0]
    onehot = (an == jax.lax.broadcasted_iota(jnp.int32, (A, MAX_Z), 1)
              ).astype(_F32)
    x = _dot(onehot, emb_ref[...])

    for b in range(N_BLOCKS):
        y = _dot(x, i2fw_ref[b]) + i2fb_ref[b]
        y2l = jnp.concatenate([y, y], axis=1)[:, None, :]
        hb = filt_l1(b + 1) if b + 1 < N_BLOCKS else None
        accs = [None] * _NT
        for u in range(_NP):
            z = w_scr[b, u].reshape(A, _TI, 2 * F) * y2l
            n = A
            while n > 1:
                n //= 2
                z = z[:n] + z[n:]
            zz = z.reshape(_TI, 2 * F)
            accs[2 * u] = zz[:, :F]
            accs[2 * u + 1] = zz[:, F:]
            if hb is not None:
                filt_l2(hb, b + 1, u)
        agg = jnp.concatenate(accs, axis=0) - w_self[b] * y
        yo = jnp.tanh(_dot(agg, f2ow_ref[b]) + f2ob_ref[b])
        x = x + _dot(yo, dw_ref[b]) + db_ref[b]

    t1 = jnp.tanh(_dot(x, e1w_ref[...]) + e1b_ref[...])
    o = _dot(t1, e2w_ref[...]) + e2b_ref[...]
    out_ref[0] = jnp.sum(o, axis=0, keepdims=True)


@jax.jit
def kernel(atomic_positions, atomic_numbers, emb, fw1_W, fw1_b, fw2_W, fw2_b,
           in2f_W, in2f_b, f2out_W, f2out_b, dense_W, dense_b,
           e1_W, e1_b, e2_W, e2_b):
    C, A = atomic_positions.shape[0], atomic_positions.shape[1]
    F, G = N_FILTERS, N_GAUSS
    an3 = atomic_numbers.astype(jnp.int32).reshape(C, A, 1)

    w1a = jnp.zeros((N_BLOCKS, _NT * G, _NT * F), _F32)
    for t in range(_NT):
        w1a = w1a.at[:, G * t:G * (t + 1), F * t:F * (t + 1)].set(fw1_W)
    w1a = w1a.astype(jnp.bfloat16)
    b1a = jnp.tile(fw1_b, (1, _NT))
    w2p = jnp.zeros((N_BLOCKS, 2 * F, 2 * F), _F32)
    w2p = w2p.at[:, :F, :F].set(fw2_W).at[:, F:, F:].set(fw2_W)
    b2d = jnp.tile(fw2_b, (1, 2))
    unf = (jnp.arange(A)[:, None] // _TI ==
           jnp.arange(_NT * G)[None, :] // G).astype(_F32)
    centers = jnp.linspace(0.0, CUTOFF, G, dtype=_F32)
    ctile = jnp.tile(centers, _NT)[None, :]

    def rep(shape):
        nd = len(shape)
        return pl.BlockSpec(shape, lambda c, _n=nd: (0,) * _n)

    in_specs = [
        pl.BlockSpec((1, A, 3), lambda c: (c, 0, 0)),
        pl.BlockSpec((1, A, 1), lambda c: (c, 0, 0)),
        rep(emb.shape),
        rep(unf.shape), rep(ctile.shape),
        rep(w1a.shape), rep(b1a.shape),
        rep(w2p.shape), rep(b2d.shape),
        rep(in2f_W.shape), rep(in2f_b.shape),
        rep(f2out_W.shape), rep(f2out_b.shape),
        rep(dense_W.shape), rep(dense_b.shape),
        rep(e1_W.shape), rep(e1_b.shape),
        rep(e2_W.shape), rep(e2_b.shape),
    ]
    out = pl.pallas_call(
        _body,
        grid=(C,),
        in_specs=in_specs,
        out_specs=pl.BlockSpec((1, 1, 1), lambda c: (c, 0, 0)),
        out_shape=jax.ShapeDtypeStruct((C, 1, 1), _F32),
        scratch_shapes=[pltpu.VMEM((N_BLOCKS, _NP, _TI * A, 2 * F), _F32),
                        pltpu.VMEM((_TI * A, _NT * G), jnp.bfloat16)],
        compiler_params=pltpu.CompilerParams(
            dimension_semantics=("arbitrary",)),
    )(atomic_positions, an3, emb, unf, ctile, w1a, b1a, w2p, b2d,
      in2f_W, in2f_b, f2out_W, f2out_b, dense_W, dense_b,
      e1_W, e1_b, e2_W, e2_b)
    return out.reshape(C, 1)
